# baseline (speedup 1.0000x reference)
import jax
import jax.numpy as jnp
from jax import lax
from jax.experimental import pallas as pl
from jax.experimental.pallas import tpu as pltpu
from jax.experimental.pallas import tpu_sc as plsc

_NAMES = ["f%d" % i for i in range(26)] + ["grp"]
_NF = 27
_NPLAIN = 26
_GRP_COLS = 4
_B = 4096
_EMB = 64
_NC = 2
_NS = 16
_NW = _NC * _NS
_BPW = _B // _NW
_NBUF = 4
_LAG = 2

_S_FIELDS = list(range(13)) + [26, 27, 28, 29]
_D_FIELDS = list(range(13, 26))


def _enqueue_rows(tab, idx_row, dst, sem):
    def chunk(c, carry):
        v = idx_row[pl.ds(c * 16, 16)]
        for l in range(16):
            r = v[l]
            pltpu.async_copy(tab.at[pl.ds(r, 1)],
                             dst.at[pl.ds(c * 16 + l, 1)], sem)
        return carry

    lax.fori_loop(0, _BPW // 16, chunk, 0)


def _enqueue_rows_hbm(tab, idx_row, out, base, sem):
    def chunk(c, carry):
        v = idx_row[pl.ds(c * 16, 16)]
        for l in range(16):
            r = v[l]
            pltpu.async_copy(tab.at[pl.ds(r, 1)],
                             out.at[pl.ds(base + c * 16 + l, 1)], sem)
        return carry

    lax.fori_loop(0, _BPW // 16, chunk, 0)


def _drain(tab, dst, sem):
    pltpu.make_async_copy(tab.at[pl.ds(0, _BPW)], dst, sem).wait()


def _body(obs_hbm, *refs):
    tabs = refs[:_NF]
    outs = refs[_NF:2 * _NF]
    scratch = refs[2 * _NF:]
    idx_v = scratch[0]
    acc_v = scratch[1]
    rows = scratch[2:2 + _NBUF]
    sems = scratch[2 + _NBUF:]
    gs = sems[0:_NBUF]
    ws = sems[_NBUF:2 * _NBUF]
    dsem = sems[2 * _NBUF]

    cid = lax.axis_index("c")
    sid = lax.axis_index("s")
    wid = sid * _NC + cid
    base = wid * _BPW

    pltpu.sync_copy(obs_hbm.at[:, pl.ds(base, _BPW)], idx_v)

    def unit_tab(u):
        return tabs[u] if u < _NPLAIN else tabs[_NF - 1]

    def unit_idx(u):
        return idx_v.at[u if u < _NPLAIN else u - _NPLAIN]

    wcop_s = [None] * _NBUF
    nu = len(_S_FIELDS)
    for k in range(nu + _LAG):
        if k < nu:
            b = k % _NBUF
            us = _S_FIELDS[k]
            if wcop_s[b] is not None:
                wcop_s[b].wait()
                wcop_s[b] = None
            _enqueue_rows(unit_tab(us), unit_idx(us), rows[b], gs[b])
        if k < len(_D_FIELDS):
            ud = _D_FIELDS[k]
            _enqueue_rows_hbm(unit_tab(ud), unit_idx(ud), outs[ud], base,
                              dsem)
        d = k - _LAG
        if d >= 0:
            db = d % _NBUF
            us = _S_FIELDS[d]
            _drain(unit_tab(us), rows[db], gs[db])
            if us < _NPLAIN:
                wcop_s[db] = pltpu.async_copy(
                    rows[db], outs[us].at[pl.ds(base, _BPW)], ws[db])

    ga, gb, gc, gd = rows[1], rows[2], rows[3], rows[0]

    def _red(r, carry):
        for c in range(_EMB // 16):
            s0 = ga[r, pl.ds(c * 16, 16)]
            s1 = gb[r, pl.ds(c * 16, 16)]
            s2 = gc[r, pl.ds(c * 16, 16)]
            s3 = gd[r, pl.ds(c * 16, 16)]
            acc_v[r, pl.ds(c * 16, 16)] = (s0 + s1) + (s2 + s3)
        return carry

    lax.fori_loop(0, _BPW, _red, 0, unroll=4)

    pltpu.sync_copy(acc_v, outs[_NF - 1].at[pl.ds(base, _BPW)])

    pltpu.make_async_copy(
        tabs[0].at[pl.ds(0, len(_D_FIELDS) * _BPW)],
        outs[13].at[pl.ds(0, len(_D_FIELDS) * _BPW)], dsem).wait()

    for c in wcop_s:
        if c is not None:
            c.wait()


def kernel(observation, tables):
    obs_t = observation.T

    mesh = plsc.VectorSubcoreMesh(core_axis_name="c", subcore_axis_name="s")
    out_type = [jax.ShapeDtypeStruct((_B, _EMB), jnp.float32)] * _NF
    scratch = (
        [pltpu.VMEM((_NF, _BPW), jnp.int32),
         pltpu.VMEM((_BPW, _EMB), jnp.float32)]
        + [pltpu.VMEM((_BPW, _EMB), jnp.float32) for _ in range(_NBUF)]
        + [pltpu.SemaphoreType.DMA for _ in range(2 * _NBUF + 1)]
    )
    run = pl.kernel(_body, out_type=out_type, mesh=mesh,
                    scratch_types=scratch)
    outs = run(obs_t, *[tables[n] for n in _NAMES])
    return tuple(outs)


# speedup vs baseline: 1.2935x; 1.2935x over previous
import jax
import jax.numpy as jnp
from jax import lax
from jax.experimental import pallas as pl
from jax.experimental.pallas import tpu as pltpu
from jax.experimental.pallas import tpu_sc as plsc

_NAMES = ["f%d" % i for i in range(26)] + ["grp"]
_NF = 27
_NPLAIN = 26
_GRP_COLS = 4
_B = 4096
_EMB = 64
_NC = 2
_NS = 16
_NW = _NC * _NS
_BPW = _B // _NW
_NBUF = 4
_LAG = 2

_S_FIELDS = list(range(6, 26)) + [26, 27, 28, 29]
_D_FIELDS = list(range(6))


def _enqueue_rows(tab, idx_row, dst, sem):
    def chunk(c, carry):
        v = idx_row[pl.ds(c * 16, 16)]
        for l in range(16):
            r = v[l]
            pltpu.async_copy(tab.at[pl.ds(r, 1)],
                             dst.at[pl.ds(c * 16 + l, 1)], sem)
        return carry

    lax.fori_loop(0, _BPW // 16, chunk, 0)


def _enqueue_rows_hbm(tab, idx_row, out, base, sem):
    def chunk(c, carry):
        v = idx_row[pl.ds(c * 16, 16)]
        for l in range(16):
            r = v[l]
            pltpu.async_copy(tab.at[pl.ds(r, 1)],
                             out.at[pl.ds(base + c * 16 + l, 1)], sem)
        return carry

    lax.fori_loop(0, _BPW // 16, chunk, 0)


def _drain(tab, dst, sem):
    pltpu.make_async_copy(tab.at[pl.ds(0, _BPW)], dst, sem).wait()


def _body(obs_hbm, *refs):
    tabs = refs[:_NF]
    outs = refs[_NF:2 * _NF]
    scratch = refs[2 * _NF:]
    idx_v = scratch[0]
    acc_v = scratch[1]
    rows = scratch[2:2 + _NBUF]
    sems = scratch[2 + _NBUF:]
    gs = sems[0:_NBUF]
    ws = sems[_NBUF:2 * _NBUF]
    dsem = sems[2 * _NBUF]

    cid = lax.axis_index("c")
    sid = lax.axis_index("s")
    wid = sid * _NC + cid
    base = wid * _BPW

    pltpu.sync_copy(obs_hbm.at[:, pl.ds(base, _BPW)], idx_v)

    def unit_tab(u):
        return tabs[u] if u < _NPLAIN else tabs[_NF - 1]

    def unit_idx(u):
        return idx_v.at[u if u < _NPLAIN else u - _NPLAIN]

    wcop_s = [None] * _NBUF
    nu = len(_S_FIELDS)
    for k in range(nu + _LAG):
        if k < nu:
            b = k % _NBUF
            us = _S_FIELDS[k]
            if wcop_s[b] is not None:
                wcop_s[b].wait()
                wcop_s[b] = None
            _enqueue_rows(unit_tab(us), unit_idx(us), rows[b], gs[b])
        if k < len(_D_FIELDS):
            ud = _D_FIELDS[k]
            _enqueue_rows_hbm(unit_tab(ud), unit_idx(ud), outs[ud], base,
                              dsem)
        d = k - _LAG
        if d >= 0:
            db = d % _NBUF
            us = _S_FIELDS[d]
            _drain(unit_tab(us), rows[db], gs[db])
            if us < _NPLAIN:
                wcop_s[db] = pltpu.async_copy(
                    rows[db], outs[us].at[pl.ds(base, _BPW)], ws[db])

    ga, gb, gc, gd = rows[0], rows[1], rows[2], rows[3]

    def _red(r, carry):
        for c in range(_EMB // 16):
            s0 = ga[r, pl.ds(c * 16, 16)]
            s1 = gb[r, pl.ds(c * 16, 16)]
            s2 = gc[r, pl.ds(c * 16, 16)]
            s3 = gd[r, pl.ds(c * 16, 16)]
            acc_v[r, pl.ds(c * 16, 16)] = (s0 + s1) + (s2 + s3)
        return carry

    lax.fori_loop(0, _BPW, _red, 0, unroll=4)

    pltpu.sync_copy(acc_v, outs[_NF - 1].at[pl.ds(base, _BPW)])

    pltpu.make_async_copy(
        tabs[0].at[pl.ds(0, len(_D_FIELDS) * _BPW)],
        outs[0].at[pl.ds(0, len(_D_FIELDS) * _BPW)], dsem).wait()

    for c in wcop_s:
        if c is not None:
            c.wait()


def kernel(observation, tables):
    obs_t = observation.T

    mesh = plsc.VectorSubcoreMesh(core_axis_name="c", subcore_axis_name="s")
    out_type = [jax.ShapeDtypeStruct((_B, _EMB), jnp.float32)] * _NF
    scratch = (
        [pltpu.VMEM((_NF, _BPW), jnp.int32),
         pltpu.VMEM((_BPW, _EMB), jnp.float32)]
        + [pltpu.VMEM((_BPW, _EMB), jnp.float32) for _ in range(_NBUF)]
        + [pltpu.SemaphoreType.DMA for _ in range(2 * _NBUF + 1)]
    )
    run = pl.kernel(_body, out_type=out_type, mesh=mesh,
                    scratch_types=scratch)
    outs = run(obs_t, *[tables[n] for n in _NAMES])
    return tuple(outs)


# speedup vs baseline: 1.6981x; 1.3128x over previous
import jax
import jax.numpy as jnp
from jax import lax
from jax.experimental import pallas as pl
from jax.experimental.pallas import tpu as pltpu
from jax.experimental.pallas import tpu_sc as plsc

_NAMES = ["f%d" % i for i in range(26)] + ["grp"]
_NF = 27
_NPLAIN = 26
_GRP_COLS = 4
_B = 4096
_EMB = 64
_NC = 2
_NS = 16
_NW = _NC * _NS
_BPW = _B // _NW
_NBUF = 4
_LAG = 2


def _enqueue_field_gather(tab, idx_row, dst, sem):
    def chunk(c, carry):
        v = idx_row[pl.ds(c * 16, 16)]
        for l in range(16):
            r = v[l]
            pltpu.async_copy(tab.at[pl.ds(r, 1)],
                             dst.at[pl.ds(c * 16 + l, 1)], sem)
        return carry

    lax.fori_loop(0, _BPW // 16, chunk, 0)


def _drain(tab, dst, sem):
    pltpu.make_async_copy(tab.at[pl.ds(0, _BPW)], dst, sem).wait()


def _body(obs_hbm, *refs):
    tabs = refs[:_NF]
    outs = refs[_NF:2 * _NF]
    scratch = refs[2 * _NF:]
    idx_v = scratch[0]
    acc_v = scratch[1]
    rows = scratch[2:2 + _NBUF]
    gsems = scratch[2 + _NBUF:2 + 2 * _NBUF]
    wsems = scratch[2 + 2 * _NBUF:2 + 3 * _NBUF]

    wid = lax.axis_index("s") * _NC + lax.axis_index("c")
    base = wid * _BPW

    pltpu.sync_copy(obs_hbm.at[:, pl.ds(base, _BPW)], idx_v)

    wcop = [None] * _NBUF
    nfields = _NPLAIN + _GRP_COLS
    for f in range(nfields + _LAG):
        if f < nfields:
            b = f % _NBUF
            if wcop[b] is not None:
                wcop[b].wait()
                wcop[b] = None
            tab = tabs[f] if f < _NPLAIN else tabs[_NF - 1]
            irow = f if f < _NPLAIN else f - _NPLAIN
            _enqueue_field_gather(tab, idx_v.at[irow], rows[b], gsems[b])
        d = f - _LAG
        if d >= 0:
            db = d % _NBUF
            dtab = tabs[d] if d < _NPLAIN else tabs[_NF - 1]
            _drain(dtab, rows[db], gsems[db])
            if d < _NPLAIN:
                wcop[db] = pltpu.async_copy(rows[db],
                                            outs[d].at[pl.ds(base, _BPW)],
                                            wsems[db])

    ga = rows[26 % _NBUF]
    gb = rows[27 % _NBUF]
    gc = rows[28 % _NBUF]
    gd = rows[29 % _NBUF]

    def _red(r, carry):
        for c in range(_EMB // 16):
            s0 = ga[r, pl.ds(c * 16, 16)]
            s1 = gb[r, pl.ds(c * 16, 16)]
            s2 = gc[r, pl.ds(c * 16, 16)]
            s3 = gd[r, pl.ds(c * 16, 16)]
            acc_v[r, pl.ds(c * 16, 16)] = (s0 + s1) + (s2 + s3)
        return carry

    lax.fori_loop(0, _BPW, _red, 0, unroll=4)

    pltpu.sync_copy(acc_v, outs[_NF - 1].at[pl.ds(base, _BPW)])
    for c in wcop:
        if c is not None:
            c.wait()


def kernel(observation, tables):
    obs_t = observation.T

    mesh = plsc.VectorSubcoreMesh(core_axis_name="c", subcore_axis_name="s")
    out_type = [jax.ShapeDtypeStruct((_B, _EMB), jnp.float32)] * _NF
    scratch = (
        [pltpu.VMEM((_NF, _BPW), jnp.int32),
         pltpu.VMEM((_BPW, _EMB), jnp.float32)]
        + [pltpu.VMEM((_BPW, _EMB), jnp.float32) for _ in range(_NBUF)]
        + [pltpu.SemaphoreType.DMA for _ in range(2 * _NBUF)]
    )
    run = pl.kernel(_body, out_type=out_type, mesh=mesh,
                    scratch_types=scratch)
    outs = run(obs_t, *[tables[n] for n in _NAMES])
    return tuple(outs)
